# baseline (speedup 1.0000x reference)
import jax
import jax.numpy as jnp
from jax import lax
from jax.experimental import pallas as pl
from jax.experimental.pallas import tpu as pltpu
from jax.experimental.pallas import tpu_sc as plsc

B = 16384
WIN = 50
S = 5
D = 64
NC = 2
NS = 16
L = 16
NW = NC * NS
BPW = B // NW
NG = BPW // L
CH = 80
NCH = (L * WIN) // CH


def _sc_body(seq_hbm, item_hbm, tgt_hbm, neg_hbm, wseq_hbm, witem_hbm,
             wout_hbm, z_hbm,
             seq_all, tgt_all,
             ia0, ia1, nb0, nb1, rv0, rv1, ur0, ur1, pr0, pr1, nr0, nr1,
             p_T, zstage, sem0, sem1):
    wid = lax.axis_index("s") * NC + lax.axis_index("c")
    lane = lax.iota(jnp.int32, L)
    laneW = lane * WIN
    laneS = lane * S

    pltpu.sync_copy(seq_hbm.at[pl.ds(wid * BPW, BPW)], seq_all)
    pltpu.sync_copy(tgt_hbm.at[pl.ds(wid * BPW, BPW)], tgt_all)

    def dmas(g, ia, nb, rv, ur, pr, nr, sem):
        cs = [pltpu.make_async_copy(
                  witem_hbm.at[ia.at[pl.ds(k * CH, CH)]],
                  rv.at[pl.ds(k * CH, CH), :], sem)
              for k in range(NCH)]
        cs.append(pltpu.make_async_copy(
            wseq_hbm.at[seq_all.at[pl.ds(g * L, L)]], ur, sem))
        cs.append(pltpu.make_async_copy(
            wout_hbm.at[tgt_all.at[pl.ds(g * L, L)]], pr, sem))
        cs.append(pltpu.make_async_copy(wout_hbm.at[nb], nr, sem))
        return cs

    def fire(g, ia, nb, rv, ur, pr, nr, sem):
        b0 = pl.multiple_of(wid * BPW + g * L, L)
        pltpu.sync_copy(item_hbm.at[pl.ds(b0 * WIN, L * WIN)], ia)
        pltpu.sync_copy(neg_hbm.at[pl.ds(b0 * S, L * S)], nb)
        for c in dmas(g, ia, nb, rv, ur, pr, nr, sem):
            c.start()

    def drain(g, ia, nb, rv, ur, pr, nr, sem):
        for c in dmas(g, ia, nb, rv, ur, pr, nr, sem):
            c.wait()

    zvec = jnp.zeros((L,), jnp.float32)

    def compute(g, rv, ur, pr, nr):
        for d in range(D):
            p_T[pl.ds(d * L, L)] = zvec

        UNR = 8
        WU = 10

        def w_body(wi, ssum):
            ridx = [laneW + wi * WU + k for k in range(WU)]

            def score_d(i, ss):
                d0 = i * UNR
                for j in range(UNR):
                    dd = (lane + d0 + j) & (D - 1)
                    uu = plsc.load_gather(ur, [lane, dd])
                    ss = tuple(ss[k] + uu * plsc.load_gather(rv, [ridx[k], dd])
                               for k in range(WU))
                return ss

            ss = lax.fori_loop(0, D // UNR, score_d, (zvec,) * WU)
            es = [jnp.exp(s * 0.125) for s in ss]

            def psum_d(i, c):
                d0 = i * UNR
                for j in range(UNR):
                    dd = (lane + d0 + j) & (D - 1)
                    acc = es[0] * plsc.load_gather(rv, [ridx[0], dd])
                    for k in range(1, WU):
                        acc = acc + es[k] * plsc.load_gather(rv, [ridx[k], dd])
                    plsc.addupdate_scatter(p_T, [dd * L + lane], acc)
                return c

            lax.fori_loop(0, D // UNR, psum_d, jnp.int32(0))
            for k in range(WU):
                ssum = ssum + es[k]
            return ssum

        ssum = lax.fori_loop(0, WIN // WU, w_body, zvec)
        inv = 1.0 / ssum

        def dot_d(i, carry):
            zp, zn = carry
            d0 = i * UNR
            for j in range(UNR):
                dd = (lane + d0 + j) & (D - 1)
                pd = plsc.load_gather(p_T, [dd * L + lane])
                zp = zp + pd * plsc.load_gather(pr, [lane, dd])
                zn = tuple(zn[s] + pd * plsc.load_gather(nr, [laneS + s, dd])
                           for s in range(S))
            return zp, zn

        zp, zn = lax.fori_loop(0, D // UNR, dot_d, (zvec, (zvec,) * S))

        out_idx = lane + g * L
        plsc.store_scatter(zstage, [out_idx], zp * inv)
        for s in range(S):
            plsc.store_scatter(zstage, [out_idx + (s + 1) * BPW], zn[s] * inv)

    buf0 = (ia0, nb0, rv0, ur0, pr0, nr0, sem0)
    buf1 = (ia1, nb1, rv1, ur1, pr1, nr1, sem1)

    fire(0, *buf0)

    def pair(i, carry):
        g0 = i * 2
        fire(g0 + 1, *buf1)
        drain(g0, *buf0)
        compute(g0, buf0[2], buf0[3], buf0[4], buf0[5])

        @pl.when(g0 + 2 < NG)
        def _():
            fire(g0 + 2, *buf0)

        drain(g0 + 1, *buf1)
        compute(g0 + 1, buf1[2], buf1[3], buf1[4], buf1[5])
        return carry

    lax.fori_loop(0, NG // 2, pair, jnp.int32(0))

    for r in range(1 + S):
        pltpu.sync_copy(zstage.at[pl.ds(r * BPW, BPW)],
                        z_hbm.at[pl.ds(r * B + wid * BPW, BPW)])


_sc_kernel = pl.kernel(
    _sc_body,
    out_type=jax.ShapeDtypeStruct(((1 + S) * B,), jnp.float32),
    mesh=plsc.VectorSubcoreMesh(core_axis_name="c", subcore_axis_name="s",
                                num_cores=NC, num_subcores=NS),
    compiler_params=pltpu.CompilerParams(needs_layout_passes=False,
                                         use_tc_tiling_on_sc=False),
    scratch_types=[
        pltpu.VMEM((BPW,), jnp.int32),
        pltpu.VMEM((BPW,), jnp.int32),
        pltpu.VMEM((L * WIN,), jnp.int32),
        pltpu.VMEM((L * WIN,), jnp.int32),
        pltpu.VMEM((L * S,), jnp.int32),
        pltpu.VMEM((L * S,), jnp.int32),
        pltpu.VMEM((L * WIN, D), jnp.float32),
        pltpu.VMEM((L * WIN, D), jnp.float32),
        pltpu.VMEM((L, D), jnp.float32),
        pltpu.VMEM((L, D), jnp.float32),
        pltpu.VMEM((L, D), jnp.float32),
        pltpu.VMEM((L, D), jnp.float32),
        pltpu.VMEM((L * S, D), jnp.float32),
        pltpu.VMEM((L * S, D), jnp.float32),
        pltpu.VMEM((D * L,), jnp.float32),
        pltpu.VMEM(((1 + S) * BPW,), jnp.float32),
        pltpu.SemaphoreType.DMA,
        pltpu.SemaphoreType.DMA,
    ],
)


def _loss_body(z_ref, out_ref):
    z = z_ref[...]
    zp = z[0:1, :]
    zn = z[1:1 + S, :]
    p = 1.0 / (1.0 + jnp.exp(-zp))
    logp = jnp.maximum(jnp.log(jnp.maximum(p, 1e-12)), -100.0)
    loss_pos = jnp.mean(-logp)
    q = 1.0 / (1.0 + jnp.exp(zn))
    log1mp = jnp.maximum(jnp.log(jnp.maximum(q, 1e-12)), -100.0)
    loss_neg = jnp.mean(-log1mp)
    out_ref[...] = jnp.reshape((loss_pos + loss_neg / S) / 2.0, (1, 1))


def kernel(seq_index, item_indices, target_index, neg_indices,
           W_seq, W_item, W_out):
    zflat = _sc_kernel(seq_index, item_indices.reshape(-1), target_index,
                       neg_indices.reshape(-1), W_seq, W_item, W_out)
    loss = pl.pallas_call(
        _loss_body,
        out_shape=jax.ShapeDtypeStruct((1, 1), jnp.float32),
    )(zflat.reshape(1 + S, B))
    return loss[0, 0]


# speedup vs baseline: 1.1778x; 1.1778x over previous
import jax
import jax.numpy as jnp
from jax import lax
from jax.experimental import pallas as pl
from jax.experimental.pallas import tpu as pltpu
from jax.experimental.pallas import tpu_sc as plsc

B = 16384
WIN = 50
S = 5
D = 64
NC = 2
NS = 16
L = 16
NW = NC * NS
BPW = B // NW
NG = BPW // L
CH = 80
NCH = (L * WIN) // CH


def _sc_body(seq_hbm, item_hbm, tgt_hbm, neg_hbm, wseq_hbm, witem_hbm,
             wout_hbm, z_hbm,
             seq_all, tgt_all,
             ia0, ia1, nb0, nb1, rv0, rv1, ur0, ur1, pr0, pr1, nr0, nr1,
             p_T, zstage, sem0, sem1):
    wid = lax.axis_index("s") * NC + lax.axis_index("c")
    lane = lax.iota(jnp.int32, L)
    laneW = lane * WIN
    laneS = lane * S

    pltpu.sync_copy(seq_hbm.at[pl.ds(wid * BPW, BPW)], seq_all)
    pltpu.sync_copy(tgt_hbm.at[pl.ds(wid * BPW, BPW)], tgt_all)

    def dmas(g, ia, nb, rv, ur, pr, nr, sem):
        cs = [pltpu.make_async_copy(
                  witem_hbm.at[ia.at[pl.ds(k * CH, CH)]],
                  rv.at[pl.ds(k * CH, CH), :], sem)
              for k in range(NCH)]
        cs.append(pltpu.make_async_copy(
            wseq_hbm.at[seq_all.at[pl.ds(g * L, L)]], ur, sem))
        cs.append(pltpu.make_async_copy(
            wout_hbm.at[tgt_all.at[pl.ds(g * L, L)]], pr, sem))
        cs.append(pltpu.make_async_copy(wout_hbm.at[nb], nr, sem))
        return cs

    def fire(g, ia, nb, rv, ur, pr, nr, sem):
        b0 = pl.multiple_of(wid * BPW + g * L, L)
        pltpu.sync_copy(item_hbm.at[pl.ds(b0 * WIN, L * WIN)], ia)
        pltpu.sync_copy(neg_hbm.at[pl.ds(b0 * S, L * S)], nb)
        for c in dmas(g, ia, nb, rv, ur, pr, nr, sem):
            c.start()

    def drain(g, ia, nb, rv, ur, pr, nr, sem):
        for c in dmas(g, ia, nb, rv, ur, pr, nr, sem):
            c.wait()

    zvec = jnp.zeros((L,), jnp.float32)

    def compute(g, rv, ur, pr, nr):
        for d in range(D):
            p_T[pl.ds(d * L, L)] = zvec

        UNR = 4
        WU = 5

        def w_body(wi, ssum):
            ridx = [laneW + wi * WU + k for k in range(WU)]

            def score_d(i, ss):
                d0 = i * UNR
                for j in range(UNR):
                    dd = (lane + d0 + j) & (D - 1)
                    uu = plsc.load_gather(ur, [lane, dd])
                    ss = tuple(ss[k] + uu * plsc.load_gather(rv, [ridx[k], dd])
                               for k in range(WU))
                return ss

            ss = lax.fori_loop(0, D // UNR, score_d, (zvec,) * WU)
            es = [jnp.exp(s * 0.125) for s in ss]

            def psum_d(i, c):
                d0 = i * UNR
                for j in range(UNR):
                    dd = (lane + d0 + j) & (D - 1)
                    acc = es[0] * plsc.load_gather(rv, [ridx[0], dd])
                    for k in range(1, WU):
                        acc = acc + es[k] * plsc.load_gather(rv, [ridx[k], dd])
                    plsc.addupdate_scatter(p_T, [dd * L + lane], acc)
                return c

            lax.fori_loop(0, D // UNR, psum_d, jnp.int32(0))
            for k in range(WU):
                ssum = ssum + es[k]
            return ssum

        ssum = lax.fori_loop(0, WIN // WU, w_body, zvec)
        inv = 1.0 / ssum

        def dot_d(i, carry):
            zp, zn = carry
            d0 = i * UNR
            for j in range(UNR):
                dd = (lane + d0 + j) & (D - 1)
                pd = plsc.load_gather(p_T, [dd * L + lane])
                zp = zp + pd * plsc.load_gather(pr, [lane, dd])
                zn = tuple(zn[s] + pd * plsc.load_gather(nr, [laneS + s, dd])
                           for s in range(S))
            return zp, zn

        zp, zn = lax.fori_loop(0, D // UNR, dot_d, (zvec, (zvec,) * S))

        out_idx = lane + g * L
        plsc.store_scatter(zstage, [out_idx], zp * inv)
        for s in range(S):
            plsc.store_scatter(zstage, [out_idx + (s + 1) * BPW], zn[s] * inv)

    buf0 = (ia0, nb0, rv0, ur0, pr0, nr0, sem0)
    buf1 = (ia1, nb1, rv1, ur1, pr1, nr1, sem1)

    fire(0, *buf0)

    def pair(i, carry):
        g0 = i * 2
        fire(g0 + 1, *buf1)
        drain(g0, *buf0)
        compute(g0, buf0[2], buf0[3], buf0[4], buf0[5])

        @pl.when(g0 + 2 < NG)
        def _():
            fire(g0 + 2, *buf0)

        drain(g0 + 1, *buf1)
        compute(g0 + 1, buf1[2], buf1[3], buf1[4], buf1[5])
        return carry

    lax.fori_loop(0, NG // 2, pair, jnp.int32(0))

    for r in range(1 + S):
        pltpu.sync_copy(zstage.at[pl.ds(r * BPW, BPW)],
                        z_hbm.at[pl.ds(r * B + wid * BPW, BPW)])


_sc_kernel = pl.kernel(
    _sc_body,
    out_type=jax.ShapeDtypeStruct(((1 + S) * B,), jnp.float32),
    mesh=plsc.VectorSubcoreMesh(core_axis_name="c", subcore_axis_name="s",
                                num_cores=NC, num_subcores=NS),
    compiler_params=pltpu.CompilerParams(needs_layout_passes=False,
                                         use_tc_tiling_on_sc=False),
    scratch_types=[
        pltpu.VMEM((BPW,), jnp.int32),
        pltpu.VMEM((BPW,), jnp.int32),
        pltpu.VMEM((L * WIN,), jnp.int32),
        pltpu.VMEM((L * WIN,), jnp.int32),
        pltpu.VMEM((L * S,), jnp.int32),
        pltpu.VMEM((L * S,), jnp.int32),
        pltpu.VMEM((L * WIN, D), jnp.float32),
        pltpu.VMEM((L * WIN, D), jnp.float32),
        pltpu.VMEM((L, D), jnp.float32),
        pltpu.VMEM((L, D), jnp.float32),
        pltpu.VMEM((L, D), jnp.float32),
        pltpu.VMEM((L, D), jnp.float32),
        pltpu.VMEM((L * S, D), jnp.float32),
        pltpu.VMEM((L * S, D), jnp.float32),
        pltpu.VMEM((D * L,), jnp.float32),
        pltpu.VMEM(((1 + S) * BPW,), jnp.float32),
        pltpu.SemaphoreType.DMA,
        pltpu.SemaphoreType.DMA,
    ],
)


def _loss_body(z_ref, out_ref):
    z = z_ref[...]
    zp = z[0:1, :]
    zn = z[1:1 + S, :]
    p = 1.0 / (1.0 + jnp.exp(-zp))
    logp = jnp.maximum(jnp.log(jnp.maximum(p, 1e-12)), -100.0)
    loss_pos = jnp.mean(-logp)
    q = 1.0 / (1.0 + jnp.exp(zn))
    log1mp = jnp.maximum(jnp.log(jnp.maximum(q, 1e-12)), -100.0)
    loss_neg = jnp.mean(-log1mp)
    out_ref[...] = jnp.reshape((loss_pos + loss_neg / S) / 2.0, (1, 1))


def kernel(seq_index, item_indices, target_index, neg_indices,
           W_seq, W_item, W_out):
    zflat = _sc_kernel(seq_index, item_indices.reshape(-1), target_index,
                       neg_indices.reshape(-1), W_seq, W_item, W_out)
    loss = pl.pallas_call(
        _loss_body,
        out_shape=jax.ShapeDtypeStruct((1, 1), jnp.float32),
    )(zflat.reshape(1 + S, B))
    return loss[0, 0]
